# Initial kernel scaffold; baseline (speedup 1.0000x reference)
import jax
import jax.numpy as jnp
from jax.experimental import pallas as pl


def kernel(x, W, b):
    raise NotImplementedError("write your pallas kernel here")


# speedup vs baseline: 7.5321x; 7.5321x over previous
import functools

import jax
import jax.numpy as jnp
from jax.experimental import pallas as pl

_N_EXPERTS = 64
_K = 8


def _router_block(x_ref, w_ref, b_ref, probs_ref, mask_ref):
    x = x_ref[...]
    w = w_ref[...]
    b = b_ref[0:1, :]
    logits = jax.lax.dot_general(
        x, w,
        dimension_numbers=(((1,), (1,)), ((), ())),
        preferred_element_type=jnp.float32,
    ) + b

    m = jnp.max(logits, axis=-1, keepdims=True)
    e = jnp.exp(logits - m)
    s = jnp.sum(e, axis=-1, keepdims=True)
    probs_ref[...] = e / s

    iota = jax.lax.broadcasted_iota(jnp.int32, logits.shape, 1)
    work = logits
    mask = jnp.zeros(logits.shape, dtype=jnp.bool_)
    for _ in range(_K):
        mx = jnp.max(work, axis=-1, keepdims=True)
        cand = jnp.where(work == mx, iota, _N_EXPERTS)
        first = jnp.min(cand, axis=-1, keepdims=True)
        sel = iota == first
        mask = jnp.logical_or(mask, sel)
        work = jnp.where(sel, -jnp.inf, work)
    mask_ref[...] = mask


@jax.jit
def kernel(x, W, b):
    n_tokens, d_model = x.shape
    block_t = 512
    grid = (n_tokens // block_t,)
    b2d = jnp.broadcast_to(b[None, :], (8, _N_EXPERTS))

    probs, mask = pl.pallas_call(
        _router_block,
        grid=grid,
        in_specs=[
            pl.BlockSpec((block_t, d_model), lambda i: (i, 0)),
            pl.BlockSpec((_N_EXPERTS, d_model), lambda i: (0, 0)),
            pl.BlockSpec((8, _N_EXPERTS), lambda i: (0, 0)),
        ],
        out_specs=[
            pl.BlockSpec((block_t, _N_EXPERTS), lambda i: (i, 0)),
            pl.BlockSpec((block_t, _N_EXPERTS), lambda i: (i, 0)),
        ],
        out_shape=[
            jax.ShapeDtypeStruct((n_tokens, _N_EXPERTS), jnp.float32),
            jax.ShapeDtypeStruct((n_tokens, _N_EXPERTS), jnp.bool_),
        ],
    )(x, W, b2d)
    return (probs, mask)


# speedup vs baseline: 12.9529x; 1.7197x over previous
import jax
import jax.numpy as jnp
from jax.experimental import pallas as pl

_N_EXPERTS = 64
_K = 8


def _router_block(x_ref, w_ref, b_ref, probs_ref, mask_ref):
    x = x_ref[...]
    w = w_ref[...]
    b_col = b_ref[:, 0:1]
    logits_t = jax.lax.dot_general(
        w, x,
        dimension_numbers=(((1,), (1,)), ((), ())),
        preferred_element_type=jnp.float32,
    ) + b_col

    m = jnp.max(logits_t, axis=0, keepdims=True)
    e = jnp.exp(logits_t - m)
    s = jnp.sum(e, axis=0, keepdims=True)
    probs_t = e / s

    iota = jax.lax.broadcasted_iota(jnp.int32, logits_t.shape, 0)
    work = logits_t
    mask_t = jnp.zeros(logits_t.shape, dtype=jnp.float32)
    for _ in range(_K):
        mx = jnp.max(work, axis=0, keepdims=True)
        cand = jnp.where(work == mx, iota, _N_EXPERTS)
        first = jnp.min(cand, axis=0, keepdims=True)
        sel = iota == first
        mask_t = jnp.where(sel, 1.0, mask_t)
        work = jnp.where(sel, -jnp.inf, work)

    probs_ref[...] = probs_t.T
    mask_ref[...] = mask_t.T > 0.5


@jax.jit
def kernel(x, W, b):
    n_tokens, d_model = x.shape
    block_t = 512
    grid = (n_tokens // block_t,)
    b2d = jnp.broadcast_to(b[:, None], (_N_EXPERTS, 128))

    probs, mask = pl.pallas_call(
        _router_block,
        grid=grid,
        in_specs=[
            pl.BlockSpec((block_t, d_model), lambda i: (i, 0)),
            pl.BlockSpec((_N_EXPERTS, d_model), lambda i: (0, 0)),
            pl.BlockSpec((_N_EXPERTS, 128), lambda i: (0, 0)),
        ],
        out_specs=[
            pl.BlockSpec((block_t, _N_EXPERTS), lambda i: (i, 0)),
            pl.BlockSpec((block_t, _N_EXPERTS), lambda i: (i, 0)),
        ],
        out_shape=[
            jax.ShapeDtypeStruct((n_tokens, _N_EXPERTS), jnp.float32),
            jax.ShapeDtypeStruct((n_tokens, _N_EXPERTS), jnp.bool_),
        ],
    )(x, W, b2d)
    return (probs, mask)


# speedup vs baseline: 15.9152x; 1.2287x over previous
import jax
import jax.numpy as jnp
from jax.experimental import pallas as pl

_N_EXPERTS = 64
_K = 8


def _router_block(x_ref, w_ref, b_ref, probs_ref, mask_ref):
    x = x_ref[...]
    w = w_ref[...]
    b_col = b_ref[:, 0:1]
    logits_t = jax.lax.dot_general(
        w, x,
        dimension_numbers=(((1,), (1,)), ((), ())),
        preferred_element_type=jnp.float32,
    ) + b_col

    m = jnp.max(logits_t, axis=0, keepdims=True)
    e = jnp.exp(logits_t - m)
    s = jnp.sum(e, axis=0, keepdims=True)
    probs_t = e / s

    iota = jax.lax.broadcasted_iota(jnp.int32, logits_t.shape, 0)
    work = logits_t
    mask_t = jnp.zeros(logits_t.shape, dtype=jnp.float32)
    for _ in range(_K):
        mx = jnp.max(work, axis=0, keepdims=True)
        cand = jnp.where(work == mx, iota, _N_EXPERTS)
        first = jnp.min(cand, axis=0, keepdims=True)
        sel = iota == first
        mask_t = jnp.where(sel, 1.0, mask_t)
        work = jnp.where(sel, -jnp.inf, work)

    probs_ref[...] = probs_t.T
    mask_ref[...] = mask_t.T > 0.5


@jax.jit
def kernel(x, W, b):
    n_tokens, d_model = x.shape
    block_t = 1024
    grid = (n_tokens // block_t,)
    b2d = jnp.broadcast_to(b[:, None], (_N_EXPERTS, 128))

    probs, mask = pl.pallas_call(
        _router_block,
        grid=grid,
        in_specs=[
            pl.BlockSpec((block_t, d_model), lambda i: (i, 0)),
            pl.BlockSpec((_N_EXPERTS, d_model), lambda i: (0, 0)),
            pl.BlockSpec((_N_EXPERTS, 128), lambda i: (0, 0)),
        ],
        out_specs=[
            pl.BlockSpec((block_t, _N_EXPERTS), lambda i: (i, 0)),
            pl.BlockSpec((block_t, _N_EXPERTS), lambda i: (i, 0)),
        ],
        out_shape=[
            jax.ShapeDtypeStruct((n_tokens, _N_EXPERTS), jnp.float32),
            jax.ShapeDtypeStruct((n_tokens, _N_EXPERTS), jnp.bool_),
        ],
    )(x, W, b2d)
    return (probs, mask)


# speedup vs baseline: 17.8632x; 1.1224x over previous
import jax
import jax.numpy as jnp
from jax.experimental import pallas as pl

_N_EXPERTS = 64
_K = 8


def _router_block(x_ref, w_ref, b_ref, probs_ref, mask_ref):
    x = x_ref[...]
    w = w_ref[...]
    b_col = b_ref[:, 0:1]
    logits_t = jax.lax.dot_general(
        w, x,
        dimension_numbers=(((1,), (1,)), ((), ())),
        preferred_element_type=jnp.float32,
    ) + b_col

    m = jnp.max(logits_t, axis=0, keepdims=True)
    e = jnp.exp(logits_t - m)
    s = jnp.sum(e, axis=0, keepdims=True)
    probs_t = e / s

    iota = jax.lax.broadcasted_iota(jnp.int32, logits_t.shape, 0)
    work = logits_t
    mask_t = jnp.zeros(logits_t.shape, dtype=jnp.float32)
    for _ in range(_K):
        mx = jnp.max(work, axis=0, keepdims=True)
        cand = jnp.where(work == mx, iota, _N_EXPERTS)
        first = jnp.min(cand, axis=0, keepdims=True)
        sel = iota == first
        mask_t = jnp.where(sel, 1.0, mask_t)
        work = jnp.where(sel, -jnp.inf, work)

    probs_ref[...] = probs_t.T
    mask_ref[...] = mask_t.T > 0.5


@jax.jit
def kernel(x, W, b):
    n_tokens, d_model = x.shape
    block_t = 2048
    grid = (n_tokens // block_t,)
    b2d = jnp.broadcast_to(b[:, None], (_N_EXPERTS, 128))

    probs, mask = pl.pallas_call(
        _router_block,
        grid=grid,
        in_specs=[
            pl.BlockSpec((block_t, d_model), lambda i: (i, 0)),
            pl.BlockSpec((_N_EXPERTS, d_model), lambda i: (0, 0)),
            pl.BlockSpec((_N_EXPERTS, 128), lambda i: (0, 0)),
        ],
        out_specs=[
            pl.BlockSpec((block_t, _N_EXPERTS), lambda i: (i, 0)),
            pl.BlockSpec((block_t, _N_EXPERTS), lambda i: (i, 0)),
        ],
        out_shape=[
            jax.ShapeDtypeStruct((n_tokens, _N_EXPERTS), jnp.float32),
            jax.ShapeDtypeStruct((n_tokens, _N_EXPERTS), jnp.bool_),
        ],
    )(x, W, b2d)
    return (probs, mask)


# speedup vs baseline: 18.6851x; 1.0460x over previous
import jax
import jax.numpy as jnp
from jax.experimental import pallas as pl

_N_EXPERTS = 64
_K = 8


def _router_block(x_ref, w_ref, b_ref, probs_ref, mask_ref):
    x = x_ref[...]
    w = w_ref[...]
    b_col = b_ref[:, 0:1]
    logits_t = jax.lax.dot_general(
        w, x,
        dimension_numbers=(((1,), (1,)), ((), ())),
        preferred_element_type=jnp.float32,
    ) + b_col

    m = jnp.max(logits_t, axis=0, keepdims=True)
    e = jnp.exp(logits_t - m)
    s = jnp.sum(e, axis=0, keepdims=True)
    probs_t = e / s

    iota = jax.lax.broadcasted_iota(jnp.int32, logits_t.shape, 0)
    work = logits_t
    mask_t = jnp.zeros(logits_t.shape, dtype=jnp.float32)
    for _ in range(_K):
        mx = jnp.max(work, axis=0, keepdims=True)
        cand = jnp.where(work == mx, iota, _N_EXPERTS)
        first = jnp.min(cand, axis=0, keepdims=True)
        sel = iota == first
        mask_t = jnp.where(sel, 1.0, mask_t)
        work = jnp.where(sel, -jnp.inf, work)

    probs_ref[...] = probs_t.T
    mask_ref[...] = mask_t.T > 0.5


@jax.jit
def kernel(x, W, b):
    n_tokens, d_model = x.shape
    block_t = 4096
    grid = (n_tokens // block_t,)
    b2d = jnp.broadcast_to(b[:, None], (_N_EXPERTS, 128))

    probs, mask = pl.pallas_call(
        _router_block,
        grid=grid,
        in_specs=[
            pl.BlockSpec((block_t, d_model), lambda i: (i, 0)),
            pl.BlockSpec((_N_EXPERTS, d_model), lambda i: (0, 0)),
            pl.BlockSpec((_N_EXPERTS, 128), lambda i: (0, 0)),
        ],
        out_specs=[
            pl.BlockSpec((block_t, _N_EXPERTS), lambda i: (i, 0)),
            pl.BlockSpec((block_t, _N_EXPERTS), lambda i: (i, 0)),
        ],
        out_shape=[
            jax.ShapeDtypeStruct((n_tokens, _N_EXPERTS), jnp.float32),
            jax.ShapeDtypeStruct((n_tokens, _N_EXPERTS), jnp.bool_),
        ],
    )(x, W, b2d)
    return (probs, mask)
